# speedup vs baseline: 1.0270x; 1.0270x over previous
import functools
import math

import jax
import jax.numpy as jnp
from jax import lax
from jax.experimental import pallas as pl
from jax.experimental.pallas import tpu as pltpu
from jax.experimental.pallas import tpu_sc as plsc

_D = 128
_NC, _NS, _L = 2, 16, 16
_NW = _NC * _NS
_B = 4096 * 200
_CHUNK = 128
_NCHUNK = _B // (_NW * _CHUNK)
_SCALE = math.sqrt(float(_D))


def _emb_body(tokens_hbm, table_hbm, out_hbm, idx_v, buf_v, sem):
    wid = lax.axis_index("s") * _NC + lax.axis_index("c")
    base = wid * (_NCHUNK * _CHUNK)
    pltpu.sync_copy(tokens_hbm.at[wid], idx_v)

    def chunk_body(j, carry):
        pltpu.async_copy(table_hbm.at[idx_v.at[j]], buf_v, sem).wait()

        def row_body(r, c2):
            for p in range(_D // _L):
                sl = pl.ds(p * _L, _L)
                buf_v[r, sl] = buf_v[r, sl] * _SCALE
            return c2

        lax.fori_loop(0, _CHUNK, row_body, 0)
        pltpu.sync_copy(buf_v, out_hbm.at[pl.ds(base + j * _CHUNK, _CHUNK)])
        return carry

    lax.fori_loop(0, _NCHUNK, chunk_body, 0)


_emb_call = functools.partial(
    pl.kernel,
    out_type=jax.ShapeDtypeStruct((_B, _D), jnp.float32),
    mesh=plsc.VectorSubcoreMesh(core_axis_name="c", subcore_axis_name="s"),
    scratch_types=[
        pltpu.VMEM((_NCHUNK, _CHUNK), jnp.int32),
        pltpu.VMEM((_CHUNK, _D), jnp.float32),
        pltpu.SemaphoreType.DMA,
    ],
)(_emb_body)


def kernel(tokens, table):
    s0, s1 = tokens.shape
    idx = tokens.astype(jnp.int32).reshape(_NW, _NCHUNK, _CHUNK)
    out = _emb_call(idx, table)
    return out.reshape(s0, s1, _D)


# speedup vs baseline: 1.8377x; 1.7894x over previous
import functools
import math

import jax
import jax.numpy as jnp
from jax import lax
from jax.experimental import pallas as pl
from jax.experimental.pallas import tpu as pltpu
from jax.experimental.pallas import tpu_sc as plsc

_D = 128
_NC, _NS, _L = 2, 16, 16
_NW = _NC * _NS
_B = 4096 * 200
_CHUNK = 128
_NCHUNK = _B // (_NW * _CHUNK)
_NBUF = 4
_SCALE = math.sqrt(float(_D))


def _scale_buf(buf):
    def row_body(r, c):
        for p in range(_D // _L):
            sl = pl.ds(p * _L, _L)
            buf[r, sl] = buf[r, sl] * _SCALE
        return c

    lax.fori_loop(0, _CHUNK, row_body, 0, unroll=2)


def _emb_body(tokens_hbm, table_hbm, out_hbm, idx_v, bufs_v, gsems, wsems):
    wid = lax.axis_index("s") * _NC + lax.axis_index("c")
    base = wid * (_NCHUNK * _CHUNK)
    pltpu.sync_copy(tokens_hbm.at[wid], idx_v)

    def gather(j, b):
        pltpu.async_copy(table_hbm.at[idx_v.at[j]], bufs_v.at[b], gsems.at[b])

    def wait_gather(j, b):
        pltpu.make_async_copy(
            table_hbm.at[idx_v.at[j]], bufs_v.at[b], gsems.at[b]).wait()

    def write(j, b):
        pltpu.async_copy(
            bufs_v.at[b], out_hbm.at[pl.ds(base + j * _CHUNK, _CHUNK)],
            wsems.at[b])

    def wait_write(j, b):
        pltpu.make_async_copy(
            bufs_v.at[b], out_hbm.at[pl.ds(base + j * _CHUNK, _CHUNK)],
            wsems.at[b]).wait()

    for b in range(_NBUF):
        gather(b, b)

    def group_body(g, carry):
        j0 = g * _NBUF
        for b in range(_NBUF):
            j = j0 + b
            bp = (b - 1) % _NBUF
            jp = j - 1

            @pl.when(jnp.logical_and(jp >= 0, jp + _NBUF < _NCHUNK))
            def _():
                wait_write(jp, bp)
                gather(jp + _NBUF, bp)

            wait_gather(j, b)
            _scale_buf(bufs_v.at[b])
            write(j, b)
        return carry

    lax.fori_loop(0, _NCHUNK // _NBUF, group_body, 0)
    for b in range(_NBUF):
        j = _NCHUNK - _NBUF + b
        wait_write(j, b)


_emb_call = functools.partial(
    pl.kernel,
    out_type=jax.ShapeDtypeStruct((_B, _D), jnp.float32),
    mesh=plsc.VectorSubcoreMesh(core_axis_name="c", subcore_axis_name="s"),
    scratch_types=[
        pltpu.VMEM((_NCHUNK, _CHUNK), jnp.int32),
        pltpu.VMEM((_NBUF, _CHUNK, _D), jnp.float32),
        pltpu.SemaphoreType.DMA((_NBUF,)),
        pltpu.SemaphoreType.DMA((_NBUF,)),
    ],
)(_emb_body)


def kernel(tokens, table):
    s0, s1 = tokens.shape
    idx = tokens.astype(jnp.int32).reshape(_NW, _NCHUNK, _CHUNK)
    out = _emb_call(idx, table)
    return out.reshape(s0, s1, _D)


# speedup vs baseline: 1.8531x; 1.0084x over previous
import functools
import math

import jax
import jax.numpy as jnp
from jax import lax
from jax.experimental import pallas as pl
from jax.experimental.pallas import tpu as pltpu
from jax.experimental.pallas import tpu_sc as plsc

_D = 128
_NC, _NS, _L = 2, 16, 16
_NW = _NC * _NS
_B = 4096 * 200
_CHUNK = 128
_NCHUNK = _B // (_NW * _CHUNK)
_NBUF = 5
_LAG = 2
_SCALE = math.sqrt(float(_D))


def _scale_buf(buf):
    def row_body(r, c):
        for p in range(_D // _L):
            sl = pl.ds(p * _L, _L)
            buf[r, sl] = buf[r, sl] * _SCALE
        return c

    lax.fori_loop(0, _CHUNK, row_body, 0, unroll=4)


def _emb_body(tokens_hbm, table_hbm, out_hbm, idx_v, bufs_v, gsems, wsems):
    wid = lax.axis_index("s") * _NC + lax.axis_index("c")
    base = wid * (_NCHUNK * _CHUNK)
    pltpu.sync_copy(tokens_hbm.at[wid], idx_v)

    def gather(j, b):
        pltpu.async_copy(table_hbm.at[idx_v.at[j]], bufs_v.at[b], gsems.at[b])

    def wait_gather(j, b):
        pltpu.make_async_copy(
            table_hbm.at[idx_v.at[j]], bufs_v.at[b], gsems.at[b]).wait()

    def write(j, b):
        pltpu.async_copy(
            bufs_v.at[b], out_hbm.at[pl.ds(base + j * _CHUNK, _CHUNK)],
            wsems.at[b])

    def wait_write(j, b):
        pltpu.make_async_copy(
            bufs_v.at[b], out_hbm.at[pl.ds(base + j * _CHUNK, _CHUNK)],
            wsems.at[b]).wait()

    for b in range(_NBUF):
        gather(b, b)

    def group_body(g, carry):
        j0 = g * _NBUF
        for b in range(_NBUF):
            j = j0 + b
            bp = (b - _LAG) % _NBUF
            jp = j - _LAG

            @pl.when(jnp.logical_and(jp >= 0, jp + _NBUF < _NCHUNK))
            def _():
                wait_write(jp, bp)
                gather(jp + _NBUF, bp)

            wait_gather(j, b)
            _scale_buf(bufs_v.at[b])
            write(j, b)
        return carry

    lax.fori_loop(0, _NCHUNK // _NBUF, group_body, 0)
    for b in range(_NBUF):
        j = _NCHUNK - _NBUF + b
        wait_write(j, b)


_emb_call = functools.partial(
    pl.kernel,
    out_type=jax.ShapeDtypeStruct((_B, _D), jnp.float32),
    mesh=plsc.VectorSubcoreMesh(core_axis_name="c", subcore_axis_name="s"),
    scratch_types=[
        pltpu.VMEM((_NCHUNK, _CHUNK), jnp.int32),
        pltpu.VMEM((_NBUF, _CHUNK, _D), jnp.float32),
        pltpu.SemaphoreType.DMA((_NBUF,)),
        pltpu.SemaphoreType.DMA((_NBUF,)),
    ],
)(_emb_body)


def kernel(tokens, table):
    s0, s1 = tokens.shape
    idx = tokens.astype(jnp.int32).reshape(_NW, _NCHUNK, _CHUNK)
    out = _emb_call(idx, table)
    return out.reshape(s0, s1, _D)


# speedup vs baseline: 1.8649x; 1.0064x over previous
import functools
import math

import jax
import jax.numpy as jnp
from jax import lax
from jax.experimental import pallas as pl
from jax.experimental.pallas import tpu as pltpu
from jax.experimental.pallas import tpu_sc as plsc

_D = 128
_NC, _NS, _L = 2, 16, 16
_NW = _NC * _NS
_B = 4096 * 200
_CHUNK = 128
_NCHUNK = _B // (_NW * _CHUNK)
_NBUF = 6
_LAG = 2
_SCALE = math.sqrt(float(_D))


def _scale_buf(buf):
    def row_body(r, c):
        for p in range(_D // _L):
            sl = pl.ds(p * _L, _L)
            buf[r, sl] = buf[r, sl] * _SCALE
        return c

    lax.fori_loop(0, _CHUNK, row_body, 0, unroll=4)


def _emb_body(tokens_hbm, table_hbm, out_hbm, idx_v, bufs_v, gsems, wsems):
    wid = lax.axis_index("s") * _NC + lax.axis_index("c")
    base = wid * (_NCHUNK * _CHUNK)
    pltpu.sync_copy(tokens_hbm.at[wid], idx_v)

    def gather(j, b):
        pltpu.async_copy(table_hbm.at[idx_v.at[j]], bufs_v.at[b], gsems.at[b])

    def wait_gather(j, b):
        pltpu.make_async_copy(
            table_hbm.at[idx_v.at[j]], bufs_v.at[b], gsems.at[b]).wait()

    def write(j, b):
        pltpu.async_copy(
            bufs_v.at[b], out_hbm.at[pl.ds(base + j * _CHUNK, _CHUNK)],
            wsems.at[b])

    def wait_write(j, b):
        pltpu.make_async_copy(
            bufs_v.at[b], out_hbm.at[pl.ds(base + j * _CHUNK, _CHUNK)],
            wsems.at[b]).wait()

    for b in range(_NBUF):
        gather(b, b)

    def turn(j, b):
        bp = (b - _LAG) % _NBUF
        jp = j - _LAG

        @pl.when(jnp.logical_and(jp >= 0, jp + _NBUF < _NCHUNK))
        def _():
            wait_write(jp, bp)
            gather(jp + _NBUF, bp)

        wait_gather(j, b)
        _scale_buf(bufs_v.at[b])
        write(j, b)

    def group_body(g, carry):
        j0 = g * _NBUF
        for b in range(_NBUF):
            turn(j0 + b, b)
        return carry

    ngroups = _NCHUNK // _NBUF
    lax.fori_loop(0, ngroups, group_body, 0)
    for j in range(ngroups * _NBUF, _NCHUNK):
        turn(j, j % _NBUF)
    for j in range(_NCHUNK - _NBUF, _NCHUNK):
        wait_write(j, j % _NBUF)


_emb_call = functools.partial(
    pl.kernel,
    out_type=jax.ShapeDtypeStruct((_B, _D), jnp.float32),
    mesh=plsc.VectorSubcoreMesh(core_axis_name="c", subcore_axis_name="s"),
    scratch_types=[
        pltpu.VMEM((_NCHUNK, _CHUNK), jnp.int32),
        pltpu.VMEM((_NBUF, _CHUNK, _D), jnp.float32),
        pltpu.SemaphoreType.DMA((_NBUF,)),
        pltpu.SemaphoreType.DMA((_NBUF,)),
    ],
)(_emb_body)


def kernel(tokens, table):
    s0, s1 = tokens.shape
    idx = tokens.astype(jnp.int32).reshape(_NW, _NCHUNK, _CHUNK)
    out = _emb_call(idx, table)
    return out.reshape(s0, s1, _D)
